# Initial kernel scaffold; baseline (speedup 1.0000x reference)
import jax
import jax.numpy as jnp
from jax.experimental import pallas as pl


def kernel(x, edge_index, supermask, Wx1, bx1, Wg, a_src, a_dst, bg, Wz1, bz1):
    raise NotImplementedError("write your pallas kernel here")


# speedup vs baseline: 1.4557x; 1.4557x over previous
import functools
import jax
import jax.numpy as jnp
from jax.experimental import pallas as pl


def _final_body(u_ref, s_ref, c_ref, o_ref):
    u = u_ref[...]
    s = s_ref[...]
    acc = jnp.zeros(o_ref.shape, o_ref.dtype)
    for l in range(6):
        acc = acc + u[:, l, :] / (s[:, l:l + 1] + 1e-16)
    o_ref[...] = jax.nn.sigmoid(acc * (1.0 / 6.0) + c_ref[...])


def kernel(x, edge_index, supermask, Wx1, bx1, Wg, a_src, a_dst, bg, Wz1, bz1):
    N = x.shape[0]
    loop = jnp.arange(N, dtype=edge_index.dtype)
    src = jnp.concatenate([edge_index[0], loop])
    dst = jnp.concatenate([edge_index[1], loop])

    h0 = jax.nn.sigmoid(x @ Wx1.T + bx1)
    M = jnp.einsum('ch,lhd->lcd', Wz1, Wg)
    Usrc = jnp.einsum('lhd,lh->ld', Wg, a_src)
    Udst = jnp.einsum('lhd,lh->ld', Wg, a_dst)
    P = jnp.einsum('nd,lcd->nlc', h0, M)
    AS = jnp.einsum('nd,ld->nl', h0, Usrc)
    AD = jnp.einsum('nd,ld->nl', h0, Udst)

    e = jnp.where(AS[src] + AD[dst] >= 0, AS[src] + AD[dst],
                  0.2 * (AS[src] + AD[dst]))
    ex = jnp.exp(jnp.minimum(e, 60.0))
    S = jax.ops.segment_sum(ex, dst, num_segments=N)
    U = jax.ops.segment_sum(ex[:, :, None] * P[src], dst, num_segments=N)

    c = (bz1 + bg.mean(axis=0) @ Wz1.T)[None, :]
    S8 = jnp.pad(S, ((0, 0), (0, 2)))

    BN = 1000
    out = pl.pallas_call(
        _final_body,
        grid=(N // BN,),
        in_specs=[
            pl.BlockSpec((BN, 6, 32), lambda i: (i, 0, 0)),
            pl.BlockSpec((BN, 8), lambda i: (i, 0)),
            pl.BlockSpec((1, 32), lambda i: (0, 0)),
        ],
        out_specs=pl.BlockSpec((BN, 32), lambda i: (i, 0)),
        out_shape=jax.ShapeDtypeStruct((N, 32), jnp.float32),
    )(U, S8, c)
    return out


# speedup vs baseline: 38.5997x; 26.5166x over previous
import functools
import jax
import jax.numpy as jnp
from jax import lax
from jax.experimental import pallas as pl
from jax.experimental.pallas import tpu as pltpu
from jax.experimental.pallas import tpu_sc as plsc

NP = 10240
PAD_NODE = 10200
EP = 331776
B = 128
NBW = 81
WIN = 5120
SINK = WIN
NSLICE = WIN // 16


def _tables_body(x_ref, wx_ref, bx_ref, us_ref, ud_ref, m2_ref,
                 t0_ref, t1_ref, ad_ref):
    h = jax.nn.sigmoid(
        jnp.dot(x_ref[...], wx_ref[...].T, preferred_element_type=jnp.float32)
        + bx_ref[...])
    z16 = jnp.zeros((h.shape[0], 16), jnp.float32)
    asb = jnp.dot(h, us_ref[...].T, preferred_element_type=jnp.float32)
    adb = jnp.dot(h, ud_ref[...].T, preferred_element_type=jnp.float32)
    pb = jnp.dot(h, m2_ref[...].T, preferred_element_type=jnp.float32)
    t0_ref[...] = jnp.concatenate([asb, z16, pb[:, 0:96]], axis=1)
    t1_ref[...] = jnp.concatenate([asb, z16, pb[:, 96:192]], axis=1)
    ad_ref[...] = jnp.concatenate([adb, z16, z16, z16, z16, z16, z16, z16],
                                  axis=1)


def _final_body(u_ref, c_ref, o_ref):
    ua = u_ref[0, 0, 0] + u_ref[0, 0, 1]
    ub = u_ref[1, 0, 0] + u_ref[1, 0, 1]
    acc = jnp.zeros(o_ref.shape, jnp.float32)
    for l in range(3):
        acc = acc + ua[:, 32 * l:32 * l + 32] / (ua[:, 96 + l:97 + l] + 1e-16)
    for l in range(3):
        acc = acc + ub[:, 32 * l:32 * l + 32] / (ub[:, 99 + l:100 + l] + 1e-16)
    o_ref[...] = jax.nn.sigmoid(acc * (1.0 / 6.0) + c_ref[...])


def _take16(v, idx16):
    return lax.gather(
        v, idx16[:, None],
        lax.GatherDimensionNumbers(offset_dims=(), collapsed_slice_dims=(0,),
                                   start_index_map=(0,)),
        (1,), mode=lax.GatherScatterMode.PROMISE_IN_BOUNDS)


def _edge_body(src_hbm, dst_hbm, t0_hbm, t1_hbm, ad_hbm, out_hbm,
               src_v, dst_v, dloc_v, t_r, ad_r, sc_b, u_sh, sem0, sem1):
    c = lax.axis_index("c")
    s = lax.axis_index("s")
    w = s * 2 + c
    lanes = lax.iota(jnp.int32, 16)
    zero16 = jnp.zeros((16,), jnp.float32)
    base = s * NSLICE

    pltpu.sync_copy(src_hbm.at[w], src_v)
    pltpu.sync_copy(dst_hbm.at[w], dst_v)

    for h, t_hbm in ((0, t0_hbm), (1, t1_hbm)):
        for win in (0, 1):
            def zrow(i, car):
                def zcol(k, car2):
                    sc_b[i, pl.ds(k * 16, 16)] = zero16
                    return car2
                return lax.fori_loop(0, 8, zcol, car)
            lax.fori_loop(0, B, zrow, 0)
            for i in range(NSLICE // B):
                pltpu.sync_copy(sc_b, u_sh.at[pl.ds(base + i * B, B)])
            @pl.when(s == 0)
            def _():
                pltpu.sync_copy(sc_b, u_sh.at[pl.ds(WIN, B)])
            plsc.subcore_barrier()

            def block(j, car):
                srcj = src_v.at[j]
                cp0 = pltpu.async_copy(t_hbm.at[srcj], t_r, sem0)
                cp1 = pltpu.async_copy(ad_hbm.at[dst_v.at[j]], ad_r, sem1)

                def remap(k, car2):
                    dv = dst_v[j, pl.ds(k * 16, 16)] - (win * WIN)
                    ok = jnp.logical_and(dv >= 0, dv < WIN)
                    dloc_v[pl.ds(k * 16, 16)] = jnp.where(
                        ok, dv, jnp.full((16,), SINK, jnp.int32))
                    return car2
                lax.fori_loop(0, 8, remap, 0)
                cp0.wait()
                cp1.wait()

                def edge_scale(e, car2):
                    v = t_r[e, pl.ds(0, 16)] + ad_r[e, pl.ds(0, 16)]
                    v = jnp.where(v >= 0.0, v, v * 0.2)
                    v = jnp.minimum(v, 60.0)
                    exv = jnp.where(lanes < 6, jnp.exp(v), 0.0)
                    sc_b[e, pl.ds(96, 16)] = exv
                    for l3 in range(3):
                        bl = _take16(exv,
                                     jnp.full((16,), 3 * h + l3, jnp.int32))
                        sc_b[e, pl.ds(32 * l3, 16)] = (
                            t_r[e, pl.ds(32 + 32 * l3, 16)] * bl)
                        sc_b[e, pl.ds(32 * l3 + 16, 16)] = (
                            t_r[e, pl.ds(48 + 32 * l3, 16)] * bl)
                    return car2
                lax.fori_loop(0, B, edge_scale, 0)

                pltpu.sync_copy(sc_b, u_sh.at[dloc_v], add=True)
                return car
            lax.fori_loop(0, NBW, block, 0)

            plsc.subcore_barrier()
            pltpu.sync_copy(u_sh.at[pl.ds(base, NSLICE)],
                            out_hbm.at[h, win, c, pl.ds(base, NSLICE)])
            plsc.subcore_barrier()


_edge_kernel = functools.partial(
    pl.kernel,
    mesh=plsc.VectorSubcoreMesh(core_axis_name="c", subcore_axis_name="s"),
    out_type=jax.ShapeDtypeStruct((2, 2, 2, WIN, 128), jnp.float32),
    scratch_types=[
        pltpu.VMEM((NBW, B), jnp.int32),
        pltpu.VMEM((NBW, B), jnp.int32),
        pltpu.VMEM((B,), jnp.int32),
        pltpu.VMEM((B, 128), jnp.float32),
        pltpu.VMEM((B, 128), jnp.float32),
        pltpu.VMEM((B, 128), jnp.float32),
        pltpu.VMEM_SHARED((WIN + B, 128), jnp.float32),
        pltpu.SemaphoreType.DMA,
        pltpu.SemaphoreType.DMA,
    ],
)(_edge_body)


def kernel(x, edge_index, supermask, Wx1, bx1, Wg, a_src, a_dst, bg, Wz1, bz1):
    N = x.shape[0]
    E = edge_index.shape[1]

    m2 = jnp.einsum('ch,lhd->lcd', Wz1, Wg).reshape(192, 64)
    usrc = jnp.einsum('lhd,lh->ld', Wg, a_src)
    udst = jnp.einsum('lhd,lh->ld', Wg, a_dst)
    usrc16 = jnp.pad(usrc, ((0, 10), (0, 0)))
    udst16 = jnp.pad(udst, ((0, 10), (0, 0)))
    cvec = (bz1 + bg.mean(axis=0) @ Wz1.T)[None, :]

    xp = jnp.pad(x, ((0, NP - N), (0, 0)))

    T0, T1, AD = pl.pallas_call(
        _tables_body,
        grid=(NP // 640,),
        in_specs=[
            pl.BlockSpec((640, 128), lambda i: (i, 0)),
            pl.BlockSpec((64, 128), lambda i: (0, 0)),
            pl.BlockSpec((1, 64), lambda i: (0, 0)),
            pl.BlockSpec((16, 64), lambda i: (0, 0)),
            pl.BlockSpec((16, 64), lambda i: (0, 0)),
            pl.BlockSpec((192, 64), lambda i: (0, 0)),
        ],
        out_specs=[
            pl.BlockSpec((640, 128), lambda i: (i, 0)),
            pl.BlockSpec((640, 128), lambda i: (i, 0)),
            pl.BlockSpec((640, 128), lambda i: (i, 0)),
        ],
        out_shape=[
            jax.ShapeDtypeStruct((NP, 128), jnp.float32),
            jax.ShapeDtypeStruct((NP, 128), jnp.float32),
            jax.ShapeDtypeStruct((NP, 128), jnp.float32),
        ],
    )(xp, Wx1, bx1[None, :], usrc16, udst16, m2)

    loop = jnp.arange(N, dtype=jnp.int32)
    padi = jnp.full((EP - E - N,), PAD_NODE, jnp.int32)
    src = jnp.concatenate([edge_index[0].astype(jnp.int32), loop, padi])
    dst = jnp.concatenate([edge_index[1].astype(jnp.int32), loop, padi])
    src = src.reshape(32, NBW, B)
    dst = dst.reshape(32, NBW, B)

    U5 = _edge_kernel(src, dst, T0, T1, AD)

    out = pl.pallas_call(
        _final_body,
        grid=(NP // 640,),
        in_specs=[
            pl.BlockSpec((2, 1, 2, 640, 128),
                         lambda i: (0, i // 8, 0, i % 8, 0)),
            pl.BlockSpec((1, 32), lambda i: (0, 0)),
        ],
        out_specs=pl.BlockSpec((640, 32), lambda i: (i, 0)),
        out_shape=jax.ShapeDtypeStruct((NP, 32), jnp.float32),
    )(U5, cvec)
    return out[:N]


# speedup vs baseline: 39.5615x; 1.0249x over previous
import functools
import jax
import jax.numpy as jnp
from jax import lax
from jax.experimental import pallas as pl
from jax.experimental.pallas import tpu as pltpu
from jax.experimental.pallas import tpu_sc as plsc

NP = 10240
PAD_NODE = 10200
EP = 331776
B = 128
NBW = 81
WIN = 5120
SINK = WIN
NSLICE = WIN // 16


def _tables_body(x_ref, wx_ref, bx_ref, us_ref, ud_ref, m2_ref,
                 t0_ref, t1_ref, ad_ref):
    h = jax.nn.sigmoid(
        jnp.dot(x_ref[...], wx_ref[...].T, preferred_element_type=jnp.float32)
        + bx_ref[...])
    z16 = jnp.zeros((h.shape[0], 16), jnp.float32)
    asb = jnp.dot(h, us_ref[...].T, preferred_element_type=jnp.float32)
    adb = jnp.dot(h, ud_ref[...].T, preferred_element_type=jnp.float32)
    pb = jnp.dot(h, m2_ref[...].T, preferred_element_type=jnp.float32)
    t0_ref[...] = jnp.concatenate([asb, z16, pb[:, 0:96]], axis=1)
    t1_ref[...] = jnp.concatenate([asb, z16, pb[:, 96:192]], axis=1)
    ad_ref[...] = jnp.concatenate([adb, z16, z16, z16, z16, z16, z16, z16],
                                  axis=1)


def _final_body(u_ref, c_ref, o_ref):
    ua = u_ref[0, 0, 0] + u_ref[0, 0, 1]
    ub = u_ref[1, 0, 0] + u_ref[1, 0, 1]
    acc = jnp.zeros(o_ref.shape, jnp.float32)
    for l in range(3):
        acc = acc + ua[:, 32 * l:32 * l + 32] / (ua[:, 96 + l:97 + l] + 1e-16)
    for l in range(3):
        acc = acc + ub[:, 32 * l:32 * l + 32] / (ub[:, 99 + l:100 + l] + 1e-16)
    o_ref[...] = jax.nn.sigmoid(acc * (1.0 / 6.0) + c_ref[...])


def _take16(v, idx16):
    return lax.gather(
        v, idx16[:, None],
        lax.GatherDimensionNumbers(offset_dims=(), collapsed_slice_dims=(0,),
                                   start_index_map=(0,)),
        (1,), mode=lax.GatherScatterMode.PROMISE_IN_BOUNDS)


def _edge_body(src_hbm, dst_hbm, t0_hbm, t1_hbm, ad_hbm, out_hbm,
               src_v, dst_v, dloc_v, t_r, ad_r, sc_b, u_sh, sem0, sem1):
    c = lax.axis_index("c")
    s = lax.axis_index("s")
    w = s * 2 + c
    lanes = lax.iota(jnp.int32, 16)
    zero16 = jnp.zeros((16,), jnp.float32)
    base = s * NSLICE

    pltpu.sync_copy(src_hbm.at[w], src_v)
    pltpu.sync_copy(dst_hbm.at[w], dst_v)

    for h, t_hbm in ((0, t0_hbm), (1, t1_hbm)):
        for win in (0, 1):
            def zrow(i, car):
                def zcol(k, car2):
                    sc_b[i, pl.ds(k * 16, 16)] = zero16
                    return car2
                return lax.fori_loop(0, 8, zcol, car)
            lax.fori_loop(0, B, zrow, 0)
            for i in range(NSLICE // B):
                pltpu.sync_copy(sc_b, u_sh.at[pl.ds(base + i * B, B)])
            @pl.when(s == 0)
            def _():
                pltpu.sync_copy(sc_b, u_sh.at[pl.ds(WIN, B)])
            plsc.subcore_barrier()

            def block(j, car):
                srcj = src_v.at[j]
                cp0 = pltpu.async_copy(t_hbm.at[srcj], t_r, sem0)
                cp1 = pltpu.async_copy(ad_hbm.at[dst_v.at[j]], ad_r, sem1)

                def remap(k, car2):
                    dv = dst_v[j, pl.ds(k * 16, 16)] - (win * WIN)
                    ok = jnp.logical_and(dv >= 0, dv < WIN)
                    dloc_v[pl.ds(k * 16, 16)] = jnp.where(ok, dv, SINK + lanes)
                    return car2
                lax.fori_loop(0, 8, remap, 0)
                cp0.wait()
                cp1.wait()

                def edge_scale(e, car2):
                    v = t_r[e, pl.ds(0, 16)] + ad_r[e, pl.ds(0, 16)]
                    v = jnp.where(v >= 0.0, v, v * 0.2)
                    v = jnp.minimum(v, 60.0)
                    exv = jnp.where(lanes < 6, jnp.exp(v), 0.0)
                    sc_b[e, pl.ds(96, 16)] = exv
                    for l3 in range(3):
                        bl = _take16(exv,
                                     jnp.full((16,), 3 * h + l3, jnp.int32))
                        sc_b[e, pl.ds(32 * l3, 16)] = (
                            t_r[e, pl.ds(32 + 32 * l3, 16)] * bl)
                        sc_b[e, pl.ds(32 * l3 + 16, 16)] = (
                            t_r[e, pl.ds(48 + 32 * l3, 16)] * bl)
                    return car2
                lax.fori_loop(0, B, edge_scale, 0)

                pltpu.sync_copy(sc_b, u_sh.at[dloc_v], add=True)
                return car
            lax.fori_loop(0, NBW, block, 0)

            plsc.subcore_barrier()
            pltpu.sync_copy(u_sh.at[pl.ds(base, NSLICE)],
                            out_hbm.at[h, win, c, pl.ds(base, NSLICE)])
            plsc.subcore_barrier()


_edge_kernel = functools.partial(
    pl.kernel,
    mesh=plsc.VectorSubcoreMesh(core_axis_name="c", subcore_axis_name="s"),
    out_type=jax.ShapeDtypeStruct((2, 2, 2, WIN, 128), jnp.float32),
    scratch_types=[
        pltpu.VMEM((NBW, B), jnp.int32),
        pltpu.VMEM((NBW, B), jnp.int32),
        pltpu.VMEM((B,), jnp.int32),
        pltpu.VMEM((B, 128), jnp.float32),
        pltpu.VMEM((B, 128), jnp.float32),
        pltpu.VMEM((B, 128), jnp.float32),
        pltpu.VMEM_SHARED((WIN + B, 128), jnp.float32),
        pltpu.SemaphoreType.DMA,
        pltpu.SemaphoreType.DMA,
    ],
)(_edge_body)


def kernel(x, edge_index, supermask, Wx1, bx1, Wg, a_src, a_dst, bg, Wz1, bz1):
    N = x.shape[0]
    E = edge_index.shape[1]

    m2 = jnp.einsum('ch,lhd->lcd', Wz1, Wg).reshape(192, 64)
    usrc = jnp.einsum('lhd,lh->ld', Wg, a_src)
    udst = jnp.einsum('lhd,lh->ld', Wg, a_dst)
    usrc16 = jnp.pad(usrc, ((0, 10), (0, 0)))
    udst16 = jnp.pad(udst, ((0, 10), (0, 0)))
    cvec = (bz1 + bg.mean(axis=0) @ Wz1.T)[None, :]

    xp = jnp.pad(x, ((0, NP - N), (0, 0)))

    T0, T1, AD = pl.pallas_call(
        _tables_body,
        grid=(NP // 640,),
        in_specs=[
            pl.BlockSpec((640, 128), lambda i: (i, 0)),
            pl.BlockSpec((64, 128), lambda i: (0, 0)),
            pl.BlockSpec((1, 64), lambda i: (0, 0)),
            pl.BlockSpec((16, 64), lambda i: (0, 0)),
            pl.BlockSpec((16, 64), lambda i: (0, 0)),
            pl.BlockSpec((192, 64), lambda i: (0, 0)),
        ],
        out_specs=[
            pl.BlockSpec((640, 128), lambda i: (i, 0)),
            pl.BlockSpec((640, 128), lambda i: (i, 0)),
            pl.BlockSpec((640, 128), lambda i: (i, 0)),
        ],
        out_shape=[
            jax.ShapeDtypeStruct((NP, 128), jnp.float32),
            jax.ShapeDtypeStruct((NP, 128), jnp.float32),
            jax.ShapeDtypeStruct((NP, 128), jnp.float32),
        ],
    )(xp, Wx1, bx1[None, :], usrc16, udst16, m2)

    loop = jnp.arange(N, dtype=jnp.int32)
    padi = jnp.full((EP - E - N,), PAD_NODE, jnp.int32)
    src = jnp.concatenate([edge_index[0].astype(jnp.int32), loop, padi])
    dst = jnp.concatenate([edge_index[1].astype(jnp.int32), loop, padi])
    src = src.reshape(32, NBW, B)
    dst = dst.reshape(32, NBW, B)

    U5 = _edge_kernel(src, dst, T0, T1, AD)

    out = pl.pallas_call(
        _final_body,
        grid=(NP // 640,),
        in_specs=[
            pl.BlockSpec((2, 1, 2, 640, 128),
                         lambda i: (0, i // 8, 0, i % 8, 0)),
            pl.BlockSpec((1, 32), lambda i: (0, 0)),
        ],
        out_specs=pl.BlockSpec((640, 32), lambda i: (i, 0)),
        out_shape=jax.ShapeDtypeStruct((NP, 32), jnp.float32),
    )(U5, cvec)
    return out[:N]


# speedup vs baseline: 39.5622x; 1.0000x over previous
import functools
import jax
import jax.numpy as jnp
from jax import lax
from jax.experimental import pallas as pl
from jax.experimental.pallas import tpu as pltpu
from jax.experimental.pallas import tpu_sc as plsc

NP = 10240
PAD_NODE = 10200
EP = 331776
B = 128
NBW = 81
WIN = 5120
SINK = WIN
NSLICE = WIN // 16


def _tables_body(x_ref, wx_ref, bx_ref, us_ref, ud_ref, m2_ref,
                 t0_ref, t1_ref, ad_ref):
    h = jax.nn.sigmoid(
        jnp.dot(x_ref[...], wx_ref[...].T, preferred_element_type=jnp.float32)
        + bx_ref[...])
    z16 = jnp.zeros((h.shape[0], 16), jnp.float32)
    asb = jnp.dot(h, us_ref[...].T, preferred_element_type=jnp.float32)
    adb = jnp.dot(h, ud_ref[...].T, preferred_element_type=jnp.float32)
    pb = jnp.dot(h, m2_ref[...].T, preferred_element_type=jnp.float32)
    t0_ref[...] = jnp.concatenate([asb, z16, pb[:, 0:96]], axis=1)
    t1_ref[...] = jnp.concatenate([asb, z16, pb[:, 96:192]], axis=1)
    ad_ref[...] = jnp.concatenate([adb, z16, z16, z16, z16, z16, z16, z16],
                                  axis=1)


def _final_body(u_ref, c_ref, o_ref):
    ua = u_ref[0, 0, 0] + u_ref[0, 0, 1]
    ub = u_ref[1, 0, 0] + u_ref[1, 0, 1]
    acc = jnp.zeros(o_ref.shape, jnp.float32)
    for l in range(3):
        acc = acc + ua[:, 32 * l:32 * l + 32] / (ua[:, 96 + l:97 + l] + 1e-16)
    for l in range(3):
        acc = acc + ub[:, 32 * l:32 * l + 32] / (ub[:, 99 + l:100 + l] + 1e-16)
    o_ref[...] = jax.nn.sigmoid(acc * (1.0 / 6.0) + c_ref[...])


def _take16(v, idx16):
    return lax.gather(
        v, idx16[:, None],
        lax.GatherDimensionNumbers(offset_dims=(), collapsed_slice_dims=(0,),
                                   start_index_map=(0,)),
        (1,), mode=lax.GatherScatterMode.PROMISE_IN_BOUNDS)


def _edge_body(src_hbm, dst_hbm, t0_hbm, t1_hbm, ad_hbm, out_hbm,
               src_v, dst_v, dloc_v, t_r, ad_r, sc_b, u_sh, sem0, sem1):
    c = lax.axis_index("c")
    s = lax.axis_index("s")
    w = s * 2 + c
    lanes = lax.iota(jnp.int32, 16)
    zero16 = jnp.zeros((16,), jnp.float32)
    base = s * NSLICE

    pltpu.sync_copy(src_hbm.at[w], src_v)
    pltpu.sync_copy(dst_hbm.at[w], dst_v)

    for h, t_hbm in ((0, t0_hbm), (1, t1_hbm)):
        for win in (0, 1):
            def zrow(i, car):
                def zcol(k, car2):
                    sc_b[i, pl.ds(k * 16, 16)] = zero16
                    return car2
                return lax.fori_loop(0, 8, zcol, car)
            lax.fori_loop(0, B, zrow, 0)
            pltpu.sync_copy(sc_b, u_sh.at[pl.ds(base, B)])
            pltpu.sync_copy(sc_b, u_sh.at[pl.ds(base + B, B)])
            pltpu.sync_copy(sc_b.at[pl.ds(0, NSLICE - 2 * B)],
                            u_sh.at[pl.ds(base + 2 * B, NSLICE - 2 * B)])
            @pl.when(s == 0)
            def _():
                pltpu.sync_copy(sc_b, u_sh.at[pl.ds(WIN, B)])
            plsc.subcore_barrier()

            def block(j, car):
                srcj = src_v.at[j]
                cp0 = pltpu.async_copy(t_hbm.at[srcj], t_r, sem0)
                cp1 = pltpu.async_copy(ad_hbm.at[dst_v.at[j]], ad_r, sem1)

                def remap(k, car2):
                    dv = dst_v[j, pl.ds(k * 16, 16)] - (win * WIN)
                    ok = jnp.logical_and(dv >= 0, dv < WIN)
                    dloc_v[pl.ds(k * 16, 16)] = jnp.where(ok, dv, SINK + lanes)
                    return car2
                lax.fori_loop(0, 8, remap, 0)
                cp0.wait()
                cp1.wait()

                def edge_scale(e, car2):
                    v = t_r[e, pl.ds(0, 16)] + ad_r[e, pl.ds(0, 16)]
                    v = jnp.where(v >= 0.0, v, v * 0.2)
                    v = jnp.minimum(v, 60.0)
                    exv = jnp.where(lanes < 6, jnp.exp(v), 0.0)
                    sc_b[e, pl.ds(96, 16)] = exv
                    for l3 in range(3):
                        bl = _take16(exv,
                                     jnp.full((16,), 3 * h + l3, jnp.int32))
                        sc_b[e, pl.ds(32 * l3, 16)] = (
                            t_r[e, pl.ds(32 + 32 * l3, 16)] * bl)
                        sc_b[e, pl.ds(32 * l3 + 16, 16)] = (
                            t_r[e, pl.ds(48 + 32 * l3, 16)] * bl)
                    return car2
                lax.fori_loop(0, B, edge_scale, 0)

                pltpu.sync_copy(sc_b, u_sh.at[dloc_v], add=True)
                return car
            lax.fori_loop(0, NBW, block, 0)

            plsc.subcore_barrier()
            pltpu.sync_copy(u_sh.at[pl.ds(base, NSLICE)],
                            out_hbm.at[h, win, c, pl.ds(base, NSLICE)])
            plsc.subcore_barrier()


_edge_kernel = functools.partial(
    pl.kernel,
    mesh=plsc.VectorSubcoreMesh(core_axis_name="c", subcore_axis_name="s"),
    out_type=jax.ShapeDtypeStruct((2, 2, 2, WIN, 128), jnp.float32),
    scratch_types=[
        pltpu.VMEM((NBW, B), jnp.int32),
        pltpu.VMEM((NBW, B), jnp.int32),
        pltpu.VMEM((B,), jnp.int32),
        pltpu.VMEM((B, 128), jnp.float32),
        pltpu.VMEM((B, 128), jnp.float32),
        pltpu.VMEM((B, 128), jnp.float32),
        pltpu.VMEM_SHARED((WIN + B, 128), jnp.float32),
        pltpu.SemaphoreType.DMA,
        pltpu.SemaphoreType.DMA,
    ],
)(_edge_body)


def kernel(x, edge_index, supermask, Wx1, bx1, Wg, a_src, a_dst, bg, Wz1, bz1):
    N = x.shape[0]
    E = edge_index.shape[1]

    m2 = jnp.einsum('ch,lhd->lcd', Wz1, Wg).reshape(192, 64)
    usrc = jnp.einsum('lhd,lh->ld', Wg, a_src)
    udst = jnp.einsum('lhd,lh->ld', Wg, a_dst)
    usrc16 = jnp.pad(usrc, ((0, 10), (0, 0)))
    udst16 = jnp.pad(udst, ((0, 10), (0, 0)))
    cvec = (bz1 + bg.mean(axis=0) @ Wz1.T)[None, :]

    xp = jnp.pad(x, ((0, NP - N), (0, 0)))

    T0, T1, AD = pl.pallas_call(
        _tables_body,
        grid=(NP // 640,),
        in_specs=[
            pl.BlockSpec((640, 128), lambda i: (i, 0)),
            pl.BlockSpec((64, 128), lambda i: (0, 0)),
            pl.BlockSpec((1, 64), lambda i: (0, 0)),
            pl.BlockSpec((16, 64), lambda i: (0, 0)),
            pl.BlockSpec((16, 64), lambda i: (0, 0)),
            pl.BlockSpec((192, 64), lambda i: (0, 0)),
        ],
        out_specs=[
            pl.BlockSpec((640, 128), lambda i: (i, 0)),
            pl.BlockSpec((640, 128), lambda i: (i, 0)),
            pl.BlockSpec((640, 128), lambda i: (i, 0)),
        ],
        out_shape=[
            jax.ShapeDtypeStruct((NP, 128), jnp.float32),
            jax.ShapeDtypeStruct((NP, 128), jnp.float32),
            jax.ShapeDtypeStruct((NP, 128), jnp.float32),
        ],
    )(xp, Wx1, bx1[None, :], usrc16, udst16, m2)

    loop = jnp.arange(N, dtype=jnp.int32)
    padi = jnp.full((EP - E - N,), PAD_NODE, jnp.int32)
    src = jnp.concatenate([edge_index[0].astype(jnp.int32), loop, padi])
    dst = jnp.concatenate([edge_index[1].astype(jnp.int32), loop, padi])
    src = src.reshape(32, NBW, B)
    dst = dst.reshape(32, NBW, B)

    U5 = _edge_kernel(src, dst, T0, T1, AD)

    out = pl.pallas_call(
        _final_body,
        grid=(NP // 640,),
        in_specs=[
            pl.BlockSpec((2, 1, 2, 640, 128),
                         lambda i: (0, i // 8, 0, i % 8, 0)),
            pl.BlockSpec((1, 32), lambda i: (0, 0)),
        ],
        out_specs=pl.BlockSpec((640, 32), lambda i: (i, 0)),
        out_shape=jax.ShapeDtypeStruct((NP, 32), jnp.float32),
    )(U5, cvec)
    return out[:N]
